# Initial kernel scaffold; baseline (speedup 1.0000x reference)
import jax
import jax.numpy as jnp
from jax.experimental import pallas as pl


def kernel(x, table):
    raise NotImplementedError("write your pallas kernel here")


# speedup vs baseline: 1.7181x; 1.7181x over previous
import jax
import jax.numpy as jnp
from jax.experimental import pallas as pl
from jax.experimental.pallas import tpu as pltpu


_VAR_MIN = 0
_PACK = 2


def _embed_kernel(idx_ref, tab_ref, out_ref):
    idx = idx_ref[...]
    t = idx.shape[0]
    r = tab_ref.shape[0] // _PACK
    iota_r = jax.lax.broadcasted_iota(jnp.int32, (t, r), 1)
    oh0 = jnp.where(idx[:, 0:1] == iota_r, 1.0, 0.0).astype(tab_ref.dtype)
    oh1 = jnp.where(idx[:, 1:2] == iota_r, 1.0, 0.0).astype(tab_ref.dtype)
    one_hot = jnp.concatenate([oh0, oh1], axis=1)
    out_ref[...] = jnp.dot(
        one_hot, tab_ref[...], preferred_element_type=jnp.float32
    ).astype(out_ref.dtype)


def kernel(x, table):
    B, S = x.shape
    R, E = table.shape
    assert E == 128 and R <= 128, "kernel specialized for E == 128, R <= 128"
    N = B * S

    tile = 4096
    n_rows = -(-N // _PACK)
    n_blocks = -(-n_rows // tile)
    padded_rows = n_blocks * tile

    idx = (x.astype(jnp.int32) - jnp.int32(_VAR_MIN)).reshape(-1)
    total = padded_rows * _PACK
    if total > N:
        idx = jnp.pad(idx, (0, total - N))
    idx_packed = idx.reshape(padded_rows, _PACK)

    tab2 = jnp.kron(jnp.eye(_PACK, dtype=table.dtype), table)

    out = pl.pallas_call(
        _embed_kernel,
        out_shape=jax.ShapeDtypeStruct((padded_rows, _PACK * E), table.dtype),
        grid=(n_blocks,),
        in_specs=[
            pl.BlockSpec((tile, _PACK), lambda i: (i, 0)),
            pl.BlockSpec((_PACK * R, _PACK * E), lambda i: (0, 0)),
        ],
        out_specs=pl.BlockSpec((tile, _PACK * E), lambda i: (i, 0)),
        compiler_params=pltpu.CompilerParams(
            dimension_semantics=("parallel",)),
    )(idx_packed, tab2)
    return out.reshape(-1, E)[:N].reshape(B, S, E)


# speedup vs baseline: 3.2633x; 1.8993x over previous
import jax
import jax.numpy as jnp
from jax.experimental import pallas as pl
from jax.experimental.pallas import tpu as pltpu


_VAR_MIN = 0
_PACK = 2


def _embed_kernel(idx_ref, tab_ref, out_ref):
    idx = idx_ref[...]
    t = idx.shape[0]
    r = tab_ref.shape[0] // _PACK
    e = out_ref.shape[1]
    iota_r = jax.lax.broadcasted_iota(jnp.int32, (t, r), 1)
    oh0 = jnp.where(idx[:, 0:1] == iota_r, 1.0, 0.0).astype(tab_ref.dtype)
    oh1 = jnp.where(idx[:, 1:2] == iota_r, 1.0, 0.0).astype(tab_ref.dtype)
    one_hot = jnp.concatenate([oh0, oh1], axis=1)
    res = jnp.dot(
        one_hot, tab_ref[...], preferred_element_type=jnp.float32
    ).astype(out_ref.dtype)
    out_ref[0::2, :] = res[:, :e]
    out_ref[1::2, :] = res[:, e:]


def kernel(x, table):
    B, S = x.shape
    R, E = table.shape
    assert E == 128 and R <= 128, "kernel specialized for E == 128, R <= 128"
    N = B * S

    tile = 4096
    n_rows = -(-N // _PACK)
    n_blocks = -(-n_rows // tile)
    padded_rows = n_blocks * tile

    idx = (x.astype(jnp.int32) - jnp.int32(_VAR_MIN)).reshape(-1)
    total = padded_rows * _PACK
    if total > N:
        idx = jnp.pad(idx, (0, total - N))
    idx_packed = idx.reshape(padded_rows, _PACK)

    tab2 = jnp.kron(jnp.eye(_PACK, dtype=table.dtype), table)

    out = pl.pallas_call(
        _embed_kernel,
        out_shape=jax.ShapeDtypeStruct((padded_rows * _PACK, E), table.dtype),
        grid=(n_blocks,),
        in_specs=[
            pl.BlockSpec((tile, _PACK), lambda i: (i, 0)),
            pl.BlockSpec((_PACK * R, _PACK * E), lambda i: (0, 0)),
        ],
        out_specs=pl.BlockSpec((tile * _PACK, E), lambda i: (i, 0)),
        compiler_params=pltpu.CompilerParams(
            dimension_semantics=("parallel",)),
    )(idx_packed, tab2)
    return out[:N].reshape(B, S, E)
